# baseline (speedup 1.0000x reference)
import functools

import jax
import jax.numpy as jnp
from jax import lax
from jax.experimental import pallas as pl
from jax.experimental.pallas import tpu as pltpu
from jax.experimental.pallas import tpu_sc as plsc

_NC = 2
_NS = 16
_L = 16
_NW = _NC * _NS


def _transpose(x):
    return jax.lax.transpose(x, (1, 0))


def _pack_pairs(lo, hi):
    lo16 = lax.bitcast_convert_type(lo.astype(jnp.bfloat16), jnp.uint16)
    hi16 = lax.bitcast_convert_type(hi.astype(jnp.bfloat16), jnp.uint16)
    w = (hi16.astype(jnp.uint32) << 16) | lo16.astype(jnp.uint32)
    return lax.bitcast_convert_type(w, jnp.int32)


def _tables_tc_body(q_hbm, c_hbm, k_hbm, lm_hbm, ptab_ref, aux_ref, ltab_ref,
                    q_ref, c_ref, k_ref, lm_ref, sem):
    cp1 = pltpu.make_async_copy(q_hbm, q_ref, sem)
    cp1.start()
    cp2 = pltpu.make_async_copy(c_hbm, c_ref, sem)
    cp2.start()
    cp3 = pltpu.make_async_copy(k_hbm, k_ref, sem)
    cp3.start()
    cp4 = pltpu.make_async_copy(lm_hbm, lm_ref, sem)
    cp4.start()
    cp1.wait()
    cp2.wait()
    cp3.wait()
    cp4.wait()
    n_kf = q_ref.shape[0]
    for h in range(n_kf // 128):
        qt = _transpose(q_ref[pl.ds(128 * h, 128), :])
        ct = _transpose(c_ref[pl.ds(128 * h, 128), :])
        x = qt[0:1, :]
        y = qt[1:2, :]
        z = qt[2:3, :]
        n = jnp.maximum(jnp.sqrt(x * x + y * y + z * z), 1e-8)
        sn = jnp.sin(n) / n
        qx = x * sn
        qy = y * sn
        qz = z * sn
        qw = jnp.cos(n)
        qn = jnp.maximum(
            jnp.sqrt(qw * qw + qx * qx + qy * qy + qz * qz), 1e-12)
        qw = qw / qn
        qx = qx / qn
        qy = qy / qn
        qz = qz / qn
        tx, ty, tz = 2.0 * qx, 2.0 * qy, 2.0 * qz
        twx, twy, twz = tx * qw, ty * qw, tz * qw
        txx, txy, txz = tx * qx, ty * qx, tz * qx
        tyy, tyz = ty * qy, tz * qy
        tzz = tz * qz
        one = jnp.ones_like(qw)
        m00 = one - (tyy + tzz)
        m01 = txy - twz
        m02 = txz + twy
        m10 = txy + twz
        m11 = one - (txx + tzz)
        m12 = tyz - twx
        m20 = txz - twy
        m21 = tyz + twx
        m22 = one - (txx + tyy)
        los = jnp.concatenate(
            [m00, m02, m11, m20, m22, ct[1:2, :]], axis=0)
        his = jnp.concatenate(
            [m01, m10, m12, m21, ct[0:1, :], ct[2:3, :]], axis=0)
        words = _pack_pairs(los, his)
        ptab_ref[pl.ds(8 * h, 8), :] = jnp.concatenate(
            [words, jnp.zeros((2, 128), jnp.int32)], axis=0)

    n_mp = lm_ref.shape[0]
    for h in range(n_mp // 128):
        lt = _transpose(lm_ref[pl.ds(128 * h, 128), :])
        los = jnp.concatenate([lt[0:1, :], lt[2:3, :]], axis=0)
        his = jnp.concatenate([lt[1:2, :], jnp.zeros((1, 128),
                                                     jnp.float32)], axis=0)
        ltab_ref[pl.ds(2 * h, 2), :] = _pack_pairs(los, his)

    km = k_ref[:]
    aux_ref[:] = jnp.concatenate(
        [
            jnp.broadcast_to(km[0:1, 0:1], (1, 128)),
            jnp.broadcast_to(km[1:2, 1:2], (1, 128)),
            jnp.broadcast_to(km[0:1, 2:3], (1, 128)),
            jnp.broadcast_to(km[1:2, 2:3], (1, 128)),
            jnp.zeros((4, 128), jnp.float32),
        ],
        axis=0,
    )


def _make_tables(quats_log, camera_position, k_mat, landmarks):
    n_kf = quats_log.shape[0]
    n_mp = landmarks.shape[0]
    return pl.pallas_call(
        _tables_tc_body,
        in_specs=[
            pl.BlockSpec(memory_space=pl.ANY),
            pl.BlockSpec(memory_space=pl.ANY),
            pl.BlockSpec(memory_space=pl.ANY),
            pl.BlockSpec(memory_space=pl.ANY),
        ],
        out_shape=(
            jax.ShapeDtypeStruct((n_kf // 16, 128), jnp.int32),
            jax.ShapeDtypeStruct((8, 128), jnp.float32),
            jax.ShapeDtypeStruct((n_mp // 64, 128), jnp.int32),
        ),
        scratch_shapes=[
            pltpu.VMEM((n_kf, 3), jnp.float32),
            pltpu.VMEM((n_kf, 3), jnp.float32),
            pltpu.VMEM((3, 3), jnp.float32),
            pltpu.VMEM((n_mp, 3), jnp.float32),
            pltpu.SemaphoreType.DMA,
        ],
    )(quats_log, camera_position, k_mat, landmarks)


def _make_sc_project(m_obs, n_kf, n_mp):
    chunk = m_obs // _NW
    steps = chunk // _L
    mesh = plsc.VectorSubcoreMesh(core_axis_name="c", subcore_axis_name="s")

    @functools.partial(
        pl.kernel,
        out_type=(jax.ShapeDtypeStruct((m_obs,), jnp.float32),
                  jax.ShapeDtypeStruct((m_obs,), jnp.float32)),
        mesh=mesh,
        compiler_params=pltpu.CompilerParams(needs_layout_passes=False),
        scratch_types=[
            pltpu.VMEM((n_kf // 16, 128), jnp.int32),
            pltpu.VMEM((8, 128), jnp.float32),
            pltpu.VMEM((n_mp // 64, 128), jnp.int32),
            pltpu.VMEM((chunk,), jnp.int32),
            pltpu.VMEM((chunk,), jnp.int32),
            pltpu.VMEM((chunk,), jnp.float32),
            pltpu.VMEM((chunk,), jnp.float32),
            pltpu.SemaphoreType.DMA,
        ],
    )
    def sc_project(ptab_hbm, aux_hbm, ltab_hbm, fid_hbm, pid_hbm, u_hbm, v_hbm,
                   ptab_v, aux_v, ltab_v, fid_v, pid_v, u_v, v_v, sem):
        wid = lax.axis_index("s") * _NC + lax.axis_index("c")
        base = wid * chunk
        cp1 = pltpu.async_copy(ptab_hbm, ptab_v, sem)
        cp2 = pltpu.async_copy(aux_hbm, aux_v, sem)
        cp3 = pltpu.async_copy(ltab_hbm, ltab_v, sem)
        cp4 = pltpu.async_copy(fid_hbm.at[pl.ds(base, chunk)], fid_v, sem)
        cp5 = pltpu.async_copy(pid_hbm.at[pl.ds(base, chunk)], pid_v, sem)
        cp1.wait()
        cp2.wait()
        cp3.wait()
        cp4.wait()
        cp5.wait()

        fxv = aux_v[0, pl.ds(0, _L)]
        fyv = aux_v[1, pl.ds(0, _L)]
        cxv = aux_v[2, pl.ds(0, _L)]
        cyv = aux_v[3, pl.ds(0, _L)]
        himask = jnp.full((_L,), jnp.int32(-65536))

        def cvec(k):
            return jnp.full((_L,), k, jnp.int32)

        def lo(w):
            return plsc.bitcast(w << 16, jnp.float32)

        def hi(w):
            return plsc.bitcast(w & himask, jnp.float32)

        @plsc.parallel_loop(0, steps, unroll=4)
        def step(i):
            off = i * _L
            fid = fid_v[pl.ds(off, _L)]
            pid = pid_v[pl.ds(off, _L)]
            fhi = (fid >> 7) << 3
            flo = fid & 127
            phi = (pid >> 7) << 1
            plo = pid & 127
            w0 = plsc.load_gather(ptab_v, [fhi + cvec(0), flo])
            w1 = plsc.load_gather(ptab_v, [fhi + cvec(1), flo])
            w2 = plsc.load_gather(ptab_v, [fhi + cvec(2), flo])
            w3 = plsc.load_gather(ptab_v, [fhi + cvec(3), flo])
            w4 = plsc.load_gather(ptab_v, [fhi + cvec(4), flo])
            w5 = plsc.load_gather(ptab_v, [fhi + cvec(5), flo])
            v0 = plsc.load_gather(ltab_v, [phi + cvec(0), plo])
            v1 = plsc.load_gather(ltab_v, [phi + cvec(1), plo])
            r00, r01 = lo(w0), hi(w0)
            r02, r10 = lo(w1), hi(w1)
            r11, r12 = lo(w2), hi(w2)
            r20, r21 = lo(w3), hi(w3)
            r22, tx = lo(w4), hi(w4)
            ty, tz = lo(w5), hi(w5)
            px, py = lo(v0), hi(v0)
            pz = lo(v1)
            xc = r00 * px + r01 * py + r02 * pz + tx
            yc = r10 * px + r11 * py + r12 * pz + ty
            zc = r20 * px + r21 * py + r22 * pz + tz
            s = jnp.where(jnp.abs(zc) > 1e-8, 1.0 / zc, jnp.ones_like(zc))
            u_v[pl.ds(off, _L)] = (xc * s) * fxv + cxv
            v_v[pl.ds(off, _L)] = (yc * s) * fyv + cyv

        pltpu.sync_copy(u_v, u_hbm.at[pl.ds(base, chunk)])
        pltpu.sync_copy(v_v, v_hbm.at[pl.ds(base, chunk)])

    return sc_project


def kernel(QuatsLog, CameraPosition, Landmarks, K, frame_id, point_id):
    ptab, aux, ltab = _make_tables(QuatsLog, CameraPosition, K, Landmarks)
    m_obs = frame_id.shape[0]
    fid = frame_id.reshape(m_obs)
    pid = point_id.reshape(m_obs)
    sc_project = _make_sc_project(m_obs, QuatsLog.shape[0], Landmarks.shape[0])
    u, v = sc_project(ptab, aux, ltab, fid, pid)
    return jnp.concatenate([u.reshape(m_obs, 1), v.reshape(m_obs, 1)], axis=1)


# speedup vs baseline: 1.0038x; 1.0038x over previous
import functools

import jax
import jax.numpy as jnp
from jax import lax
from jax.experimental import pallas as pl
from jax.experimental.pallas import tpu as pltpu
from jax.experimental.pallas import tpu_sc as plsc

_NC = 2
_NS = 16
_L = 16
_NW = _NC * _NS


def _transpose(x):
    return jax.lax.transpose(x, (1, 0))


def _pack_pairs(lo, hi):
    lo16 = lax.bitcast_convert_type(lo.astype(jnp.bfloat16), jnp.uint16)
    hi16 = lax.bitcast_convert_type(hi.astype(jnp.bfloat16), jnp.uint16)
    w = (hi16.astype(jnp.uint32) << 16) | lo16.astype(jnp.uint32)
    return lax.bitcast_convert_type(w, jnp.int32)


def _tables_tc_body(q_ref, c_ref, k_ref, lm_ref, ptab_ref, aux_ref, ltab_ref):
    n_kf = q_ref.shape[0]
    for h in range(n_kf // 128):
        qt = _transpose(q_ref[pl.ds(128 * h, 128), :])
        ct = _transpose(c_ref[pl.ds(128 * h, 128), :])
        x = qt[0:1, :]
        y = qt[1:2, :]
        z = qt[2:3, :]
        n = jnp.maximum(jnp.sqrt(x * x + y * y + z * z), 1e-8)
        sn = jnp.sin(n) / n
        qx = x * sn
        qy = y * sn
        qz = z * sn
        qw = jnp.cos(n)
        qn = jnp.maximum(
            jnp.sqrt(qw * qw + qx * qx + qy * qy + qz * qz), 1e-12)
        qw = qw / qn
        qx = qx / qn
        qy = qy / qn
        qz = qz / qn
        tx, ty, tz = 2.0 * qx, 2.0 * qy, 2.0 * qz
        twx, twy, twz = tx * qw, ty * qw, tz * qw
        txx, txy, txz = tx * qx, ty * qx, tz * qx
        tyy, tyz = ty * qy, tz * qy
        tzz = tz * qz
        one = jnp.ones_like(qw)
        m00 = one - (tyy + tzz)
        m01 = txy - twz
        m02 = txz + twy
        m10 = txy + twz
        m11 = one - (txx + tzz)
        m12 = tyz - twx
        m20 = txz - twy
        m21 = tyz + twx
        m22 = one - (txx + tyy)
        los = jnp.concatenate(
            [m00, m02, m11, m20, m22, ct[1:2, :]], axis=0)
        his = jnp.concatenate(
            [m01, m10, m12, m21, ct[0:1, :], ct[2:3, :]], axis=0)
        words = _pack_pairs(los, his)
        ptab_ref[pl.ds(8 * h, 8), :] = jnp.concatenate(
            [words, jnp.zeros((2, 128), jnp.int32)], axis=0)

    n_mp = lm_ref.shape[0]
    for h in range(n_mp // 128):
        lt = _transpose(lm_ref[pl.ds(128 * h, 128), :])
        los = jnp.concatenate([lt[0:1, :], lt[2:3, :]], axis=0)
        his = jnp.concatenate([lt[1:2, :], jnp.zeros((1, 128),
                                                     jnp.float32)], axis=0)
        ltab_ref[pl.ds(2 * h, 2), :] = _pack_pairs(los, his)

    km = k_ref[:]
    aux_ref[:] = jnp.concatenate(
        [
            jnp.broadcast_to(km[0:1, 0:1], (1, 128)),
            jnp.broadcast_to(km[1:2, 1:2], (1, 128)),
            jnp.broadcast_to(km[0:1, 2:3], (1, 128)),
            jnp.broadcast_to(km[1:2, 2:3], (1, 128)),
            jnp.zeros((4, 128), jnp.float32),
        ],
        axis=0,
    )


def _make_tables(quats_log, camera_position, k_mat, landmarks):
    n_kf = quats_log.shape[0]
    n_mp = landmarks.shape[0]
    return pl.pallas_call(
        _tables_tc_body,
        out_shape=(
            jax.ShapeDtypeStruct((n_kf // 16, 128), jnp.int32),
            jax.ShapeDtypeStruct((8, 128), jnp.float32),
            jax.ShapeDtypeStruct((n_mp // 64, 128), jnp.int32),
        ),
    )(quats_log, camera_position, k_mat, landmarks)


def _make_sc_project(m_obs, n_kf, n_mp):
    chunk = m_obs // _NW
    steps = chunk // _L
    mesh = plsc.VectorSubcoreMesh(core_axis_name="c", subcore_axis_name="s")

    @functools.partial(
        pl.kernel,
        out_type=(jax.ShapeDtypeStruct((m_obs,), jnp.float32),
                  jax.ShapeDtypeStruct((m_obs,), jnp.float32)),
        mesh=mesh,
        compiler_params=pltpu.CompilerParams(needs_layout_passes=False),
        scratch_types=[
            pltpu.VMEM((n_kf // 16, 128), jnp.int32),
            pltpu.VMEM((8, 128), jnp.float32),
            pltpu.VMEM((n_mp // 64, 128), jnp.int32),
            pltpu.VMEM((chunk,), jnp.int32),
            pltpu.VMEM((chunk,), jnp.int32),
            pltpu.VMEM((chunk,), jnp.float32),
            pltpu.VMEM((chunk,), jnp.float32),
            pltpu.SemaphoreType.DMA,
        ],
    )
    def sc_project(ptab_hbm, aux_hbm, ltab_hbm, fid_hbm, pid_hbm, u_hbm, v_hbm,
                   ptab_v, aux_v, ltab_v, fid_v, pid_v, u_v, v_v, sem):
        wid = lax.axis_index("s") * _NC + lax.axis_index("c")
        base = wid * chunk
        cp1 = pltpu.async_copy(ptab_hbm, ptab_v, sem)
        cp2 = pltpu.async_copy(aux_hbm, aux_v, sem)
        cp3 = pltpu.async_copy(ltab_hbm, ltab_v, sem)
        cp4 = pltpu.async_copy(fid_hbm.at[pl.ds(base, chunk)], fid_v, sem)
        cp5 = pltpu.async_copy(pid_hbm.at[pl.ds(base, chunk)], pid_v, sem)
        cp1.wait()
        cp2.wait()
        cp3.wait()
        cp4.wait()
        cp5.wait()

        fxv = aux_v[0, pl.ds(0, _L)]
        fyv = aux_v[1, pl.ds(0, _L)]
        cxv = aux_v[2, pl.ds(0, _L)]
        cyv = aux_v[3, pl.ds(0, _L)]
        himask = jnp.full((_L,), jnp.int32(-65536))

        def cvec(k):
            return jnp.full((_L,), k, jnp.int32)

        def lo(w):
            return plsc.bitcast(w << 16, jnp.float32)

        def hi(w):
            return plsc.bitcast(w & himask, jnp.float32)

        @plsc.parallel_loop(0, steps, unroll=4)
        def step(i):
            off = i * _L
            fid = fid_v[pl.ds(off, _L)]
            pid = pid_v[pl.ds(off, _L)]
            fhi = (fid >> 7) << 3
            flo = fid & 127
            phi = (pid >> 7) << 1
            plo = pid & 127
            w0 = plsc.load_gather(ptab_v, [fhi + cvec(0), flo])
            w1 = plsc.load_gather(ptab_v, [fhi + cvec(1), flo])
            w2 = plsc.load_gather(ptab_v, [fhi + cvec(2), flo])
            w3 = plsc.load_gather(ptab_v, [fhi + cvec(3), flo])
            w4 = plsc.load_gather(ptab_v, [fhi + cvec(4), flo])
            w5 = plsc.load_gather(ptab_v, [fhi + cvec(5), flo])
            v0 = plsc.load_gather(ltab_v, [phi + cvec(0), plo])
            v1 = plsc.load_gather(ltab_v, [phi + cvec(1), plo])
            r00, r01 = lo(w0), hi(w0)
            r02, r10 = lo(w1), hi(w1)
            r11, r12 = lo(w2), hi(w2)
            r20, r21 = lo(w3), hi(w3)
            r22, tx = lo(w4), hi(w4)
            ty, tz = lo(w5), hi(w5)
            px, py = lo(v0), hi(v0)
            pz = lo(v1)
            xc = r00 * px + r01 * py + r02 * pz + tx
            yc = r10 * px + r11 * py + r12 * pz + ty
            zc = r20 * px + r21 * py + r22 * pz + tz
            s = jnp.where(jnp.abs(zc) > 1e-8, 1.0 / zc, jnp.ones_like(zc))
            u_v[pl.ds(off, _L)] = (xc * s) * fxv + cxv
            v_v[pl.ds(off, _L)] = (yc * s) * fyv + cyv

        pltpu.sync_copy(u_v, u_hbm.at[pl.ds(base, chunk)])
        pltpu.sync_copy(v_v, v_hbm.at[pl.ds(base, chunk)])

    return sc_project


def kernel(QuatsLog, CameraPosition, Landmarks, K, frame_id, point_id):
    ptab, aux, ltab = _make_tables(QuatsLog, CameraPosition, K, Landmarks)
    m_obs = frame_id.shape[0]
    fid = frame_id.reshape(m_obs)
    pid = point_id.reshape(m_obs)
    sc_project = _make_sc_project(m_obs, QuatsLog.shape[0], Landmarks.shape[0])
    u, v = sc_project(ptab, aux, ltab, fid, pid)
    return jnp.concatenate([u.reshape(m_obs, 1), v.reshape(m_obs, 1)], axis=1)
